# baseline (speedup 1.0000x reference)
import dataclasses
import functools

import jax
import jax.numpy as jnp
from jax.experimental import pallas as pl
from jax.experimental.pallas import tpu as pltpu
from jax.experimental.pallas import tpu_sc as plsc

N_TABLE = 2048
SCALE = 64.0
MAGIC = 1.5 * 2.0**23
LANES = 16

BLK_ROWS = 128


def _sc_compiler_params():
    cp = pltpu.CompilerParams()
    if "needs_layout_passes" in pltpu.CompilerParams.__dataclass_fields__:
        cp = dataclasses.replace(cp, needs_layout_passes=False)
    return cp


def kernel(x, table):
    b, m, n = x.shape
    blk_m = 8
    mesh = plsc.VectorSubcoreMesh(core_axis_name="c", subcore_axis_name="s")

    @functools.partial(
        pl.kernel,
        out_type=jax.ShapeDtypeStruct(x.shape, jnp.float32),
        mesh=mesh,
        scratch_types=[pltpu.VMEM((N_TABLE,), jnp.float32)],
        compiler_params=_sc_compiler_params(),
    )
    def _lut_kernel(x_hbm, t_hbm, o_hbm, table_v):
        pltpu.sync_copy(t_hbm, table_v)

        def body(in_v, out_v):
            @plsc.parallel_loop(0, blk_m, unroll=2)
            def _(r):
                for c in range(0, n, LANES):
                    sl = pl.ds(c, LANES)
                    v = in_v[0, r, sl]
                    biased = v * SCALE + MAGIC
                    idx = plsc.bitcast(biased, jnp.int32) & (N_TABLE - 1)
                    out_v[0, r, sl] = plsc.load_gather(table_v, [idx])

        pltpu.emit_pipeline(
            body,
            grid=(b, m // blk_m),
            in_specs=[pl.BlockSpec((1, blk_m, n), lambda i, j: (i, j, 0))],
            out_specs=[pl.BlockSpec((1, blk_m, n), lambda i, j: (i, j, 0))],
            core_axis_name=("c", "s"),
            dimension_semantics=(pltpu.PARALLEL, pltpu.PARALLEL),
        )(x_hbm, o_hbm)

    return _lut_kernel(x, table)


# speedup vs baseline: 4.5899x; 4.5899x over previous
import dataclasses
import functools

import jax
import jax.numpy as jnp
from jax.experimental import pallas as pl
from jax.experimental.pallas import tpu as pltpu
from jax.experimental.pallas import tpu_sc as plsc

N_TABLE = 2048
SCALE = 64.0
MAGIC = 1.5 * 2.0**23
LANES = 16

BLK_ROWS = 128


def _sc_compiler_params():
    cp = pltpu.CompilerParams()
    if "needs_layout_passes" in pltpu.CompilerParams.__dataclass_fields__:
        cp = dataclasses.replace(cp, needs_layout_passes=False)
    return cp


def kernel(x, table):
    b, m, n = x.shape
    blk_m = 64
    blk_n = 256
    mesh = plsc.VectorSubcoreMesh(core_axis_name="c", subcore_axis_name="s")

    @functools.partial(
        pl.kernel,
        out_type=jax.ShapeDtypeStruct(x.shape, jnp.float32),
        mesh=mesh,
        scratch_types=[pltpu.VMEM((N_TABLE,), jnp.float32)],
        compiler_params=_sc_compiler_params(),
    )
    def _lut_kernel(x_hbm, t_hbm, o_hbm, table_v):
        pltpu.sync_copy(t_hbm, table_v)

        def body(in_v, out_v):
            @plsc.parallel_loop(0, blk_m, unroll=2)
            def _(r):
                for c in range(0, blk_n, LANES):
                    sl = pl.ds(c, LANES)
                    v = in_v[0, r, sl]
                    biased = v * SCALE + MAGIC
                    idx = plsc.bitcast(biased, jnp.int32) & (N_TABLE - 1)
                    out_v[0, r, sl] = plsc.load_gather(table_v, [idx])

        pltpu.emit_pipeline(
            body,
            grid=(b, m // blk_m, n // blk_n),
            in_specs=[pl.BlockSpec((1, blk_m, blk_n), lambda i, j, k: (i, j, k))],
            out_specs=[pl.BlockSpec((1, blk_m, blk_n), lambda i, j, k: (i, j, k))],
            core_axis_name=("c", "s"),
            dimension_semantics=(pltpu.PARALLEL, pltpu.PARALLEL, pltpu.PARALLEL),
        )(x_hbm, o_hbm)

    return _lut_kernel(x, table)


# speedup vs baseline: 6.0351x; 1.3149x over previous
import dataclasses
import functools

import jax
import jax.numpy as jnp
from jax.experimental import pallas as pl
from jax.experimental.pallas import tpu as pltpu
from jax.experimental.pallas import tpu_sc as plsc

N_TABLE = 2048
SCALE = 64.0
MAGIC = 1.5 * 2.0**23
LANES = 16

BLK_ROWS = 128


def _sc_compiler_params():
    cp = pltpu.CompilerParams()
    if "needs_layout_passes" in pltpu.CompilerParams.__dataclass_fields__:
        cp = dataclasses.replace(cp, needs_layout_passes=False)
    return cp


def kernel(x, table):
    b, m, n = x.shape
    blk_m = 8
    blk_n = 2048
    mesh = plsc.VectorSubcoreMesh(core_axis_name="c", subcore_axis_name="s")

    @functools.partial(
        pl.kernel,
        out_type=jax.ShapeDtypeStruct(x.shape, jnp.float32),
        mesh=mesh,
        scratch_types=[pltpu.VMEM((N_TABLE,), jnp.float32)],
        compiler_params=_sc_compiler_params(),
    )
    def _lut_kernel(x_hbm, t_hbm, o_hbm, table_v):
        pltpu.sync_copy(t_hbm, table_v)

        def body(in_v, out_v):
            @plsc.parallel_loop(0, blk_m * blk_n, step=8 * LANES, unroll=8)
            def _(i):
                r = jax.lax.shift_right_logical(i, 11)
                base = jax.lax.bitwise_and(i, blk_n - 1)
                for k in range(8):
                    sl = pl.ds(base + k * LANES, LANES)
                    v = in_v[0, r, sl]
                    biased = v * SCALE + MAGIC
                    idx = plsc.bitcast(biased, jnp.int32) & (N_TABLE - 1)
                    out_v[0, r, sl] = plsc.load_gather(table_v, [idx])

        pltpu.emit_pipeline(
            body,
            grid=(b, m // blk_m, n // blk_n),
            in_specs=[pl.BlockSpec((1, blk_m, blk_n), lambda i, j, k: (i, j, k))],
            out_specs=[pl.BlockSpec((1, blk_m, blk_n), lambda i, j, k: (i, j, k))],
            core_axis_name=("c", "s"),
            dimension_semantics=(pltpu.PARALLEL, pltpu.PARALLEL, pltpu.PARALLEL),
            trace_scopes=False,
        )(x_hbm, o_hbm)

    return _lut_kernel(x, table)


# speedup vs baseline: 6.0949x; 1.0099x over previous
import dataclasses
import functools

import jax
import jax.numpy as jnp
from jax.experimental import pallas as pl
from jax.experimental.pallas import tpu as pltpu
from jax.experimental.pallas import tpu_sc as plsc

N_TABLE = 2048
SCALE = 64.0
MAGIC = 1.5 * 2.0**23
LANES = 16

BLK_ROWS = 128


def _sc_compiler_params():
    cp = pltpu.CompilerParams()
    if "needs_layout_passes" in pltpu.CompilerParams.__dataclass_fields__:
        cp = dataclasses.replace(cp, needs_layout_passes=False)
    return cp


def kernel(x, table):
    b, m, n = x.shape
    blk_m = 8
    blk_n = 2048
    mesh = plsc.VectorSubcoreMesh(core_axis_name="c", subcore_axis_name="s")

    table_rep = jnp.repeat(table, LANES)

    @functools.partial(
        pl.kernel,
        out_type=jax.ShapeDtypeStruct(x.shape, jnp.float32),
        mesh=mesh,
        scratch_types=[pltpu.VMEM((N_TABLE * LANES,), jnp.float32)],
        compiler_params=_sc_compiler_params(),
    )
    def _lut_kernel(x_hbm, t_hbm, o_hbm, table_v):
        pltpu.sync_copy(t_hbm, table_v)

        def body(in_v, out_v):
            lane = jax.lax.iota(jnp.int32, LANES)

            @plsc.parallel_loop(0, blk_m * blk_n, step=8 * LANES, unroll=8)
            def _(i):
                r = jax.lax.shift_right_logical(i, 11)
                base = jax.lax.bitwise_and(i, blk_n - 1)
                for k in range(8):
                    sl = pl.ds(base + k * LANES, LANES)
                    v = in_v[0, r, sl]
                    biased = v * SCALE + MAGIC
                    shifted = jax.lax.shift_left(plsc.bitcast(biased, jnp.int32), 4)
                    idx = (shifted & (N_TABLE * LANES - 1)) | lane
                    out_v[0, r, sl] = plsc.load_gather(table_v, [idx])

        pltpu.emit_pipeline(
            body,
            grid=(b, m // blk_m, n // blk_n),
            in_specs=[pl.BlockSpec((1, blk_m, blk_n), lambda i, j, k: (i, j, k))],
            out_specs=[pl.BlockSpec((1, blk_m, blk_n), lambda i, j, k: (i, j, k))],
            core_axis_name=("c", "s"),
            dimension_semantics=(pltpu.PARALLEL, pltpu.PARALLEL, pltpu.PARALLEL),
            trace_scopes=False,
        )(x_hbm, o_hbm)

    return _lut_kernel(x, table_rep)
